# baseline (speedup 1.0000x reference)
import jax
import jax.numpy as jnp
from jax import lax
from jax.experimental import pallas as pl
from jax.experimental.pallas import tpu as pltpu
from jax.experimental.pallas import tpu_sc as plsc

_VOCAB = 1000
_VP = 1024
_EMB = 64
_B = 16384
_L = 200
_H1 = 256
_H2 = 256
_OUT = 128

_NW = 32
_ROWS_PER_W = _B // _NW
_CHUNK = 64
_NCHUNK = _ROWS_PER_W // _CHUNK
_JFULL = _L // 16


_WP = _VP // 4


def _make_counts_call(nrows):
    rows_per_w = nrows // _NW
    nchunk = rows_per_w // _CHUNK

    def body(x_hbm, cnt_hbm, x_v0, x_v1, cnt_v0, cnt_v1,
             sx0, sx1, sc0, sc1):
        wid = lax.axis_index("s") * 2 + lax.axis_index("c")
        lanes = lax.iota(jnp.int32, 16)
        tail_mask = lanes >= 8
        one16 = jnp.full((16,), 1, jnp.int32)
        zero16 = jnp.zeros((16,), jnp.int32)
        x_bufs, cnt_bufs = [x_v0, x_v1], [cnt_v0, cnt_v1]
        x_sems, cnt_sems = [sx0, sx1], [sc0, sc1]

        def scat(cv, xv, rvec, mask=None):
            col = jnp.bitwise_and(xv, 255)
            sh = jnp.right_shift(xv, 5) & 24
            val = jnp.left_shift(one16, sh)
            plsc.addupdate_scatter(cv, [rvec, col], val, mask=mask)

        def rowbase(c):
            return wid * rows_per_w + c * _CHUNK

        x_pend = [None, None]
        cnt_pend = [None, None]
        x_pend[0] = pltpu.async_copy(
            x_hbm.at[pl.ds(rowbase(0), _CHUNK)], x_bufs[0], x_sems[0])
        for c in range(nchunk):
            b = c % 2
            x_pend[b].wait()
            if c + 1 < nchunk:
                nb = (c + 1) % 2
                x_pend[nb] = pltpu.async_copy(
                    x_hbm.at[pl.ds(rowbase(c + 1), _CHUNK)],
                    x_bufs[nb], x_sems[nb])
            if cnt_pend[b] is not None:
                cnt_pend[b].wait()
            x_v, cnt_v = x_bufs[b], cnt_bufs[b]

            def zbody(r, carry):
                for j in range(_WP // 16):
                    cnt_v[r, pl.ds(j * 16, 16)] = zero16
                return carry

            lax.fori_loop(0, _CHUNK, zbody, 0)

            def rbody(rg, carry):
                rows = [rg * 4 + i for i in range(4)]
                rvecs = [jnp.full((16,), r, jnp.int32) for r in rows]
                for j in range(_JFULL):
                    xs = [x_v[r, pl.ds(j * 16, 16)] for r in rows]
                    for i in range(4):
                        scat(cnt_v, xs[i], rvecs[i])
                xts = [x_v[r, pl.ds(_L - 16, 16)] for r in rows]
                for i in range(4):
                    scat(cnt_v, xts[i], rvecs[i], mask=tail_mask)
                return carry

            lax.fori_loop(0, _CHUNK // 4, rbody, 0)

            cnt_pend[b] = pltpu.async_copy(
                cnt_v, cnt_hbm.at[pl.ds(rowbase(c), _CHUNK)], cnt_sems[b])
        for p in cnt_pend:
            if p is not None:
                p.wait()

    return pl.kernel(
        body,
        out_type=jax.ShapeDtypeStruct((nrows, _WP), jnp.int32),
        mesh=plsc.VectorSubcoreMesh(core_axis_name="c", subcore_axis_name="s"),
        scratch_types=[
            pltpu.VMEM((_CHUNK, _L), jnp.int32),
            pltpu.VMEM((_CHUNK, _L), jnp.int32),
            pltpu.VMEM((_CHUNK, _WP), jnp.int32),
            pltpu.VMEM((_CHUNK, _WP), jnp.int32),
            pltpu.SemaphoreType.DMA,
            pltpu.SemaphoreType.DMA,
            pltpu.SemaphoreType.DMA,
            pltpu.SemaphoreType.DMA,
        ],
        compiler_params=pltpu.CompilerParams(needs_layout_passes=False),
    )


_NSPLIT = 4
_BSPLIT = _B // _NSPLIT
_counts_half = _make_counts_call(_BSPLIT)

_BB = 512


def _mlp_body(cnt, embp, w1, b1, w2, b2, wout, bout, out, m1):
    @pl.when(pl.program_id(0) == 0)
    def _():
        m1[...] = jnp.dot(embp[...], w1[...],
                          preferred_element_type=jnp.float32) * (1.0 / _L)

    w = cnt[...]
    h = jnp.zeros((_BB, _H1), jnp.float32)
    for k in range(4):
        part = ((w >> (8 * k)) & 0xFF).astype(jnp.float32)
        h = h + jnp.dot(part, m1[pl.ds(k * _WP, _WP), :],
                        preferred_element_type=jnp.float32)
    h = jnp.maximum(h + b1[...], 0.0)
    h = jnp.maximum(
        jnp.dot(h, w2[...], preferred_element_type=jnp.float32) + b2[...], 0.0)
    out[...] = jnp.dot(h, wout[...],
                       preferred_element_type=jnp.float32) + bout[...]


_mlp_call = pl.pallas_call(
    _mlp_body,
    grid=(_BSPLIT // _BB,),
    in_specs=[
        pl.BlockSpec((_BB, _WP), lambda i: (i, 0)),
        pl.BlockSpec((_VP, _EMB), lambda i: (0, 0)),
        pl.BlockSpec((_EMB, _H1), lambda i: (0, 0)),
        pl.BlockSpec((1, _H1), lambda i: (0, 0)),
        pl.BlockSpec((_H1, _H2), lambda i: (0, 0)),
        pl.BlockSpec((1, _H2), lambda i: (0, 0)),
        pl.BlockSpec((_H2, _OUT), lambda i: (0, 0)),
        pl.BlockSpec((1, _OUT), lambda i: (0, 0)),
    ],
    out_specs=pl.BlockSpec((_BB, _OUT), lambda i: (i, 0)),
    out_shape=jax.ShapeDtypeStruct((_BSPLIT, _OUT), jnp.float32),
    scratch_shapes=[pltpu.VMEM((_VP, _H1), jnp.float32)],
)


def kernel(x, emb, W1, b1, W2, b2, Wout, bout):
    embp = jnp.pad(emb, ((0, _VP - _VOCAB), (0, 0)))
    b1r, b2r, boutr = b1.reshape(1, _H1), b2.reshape(1, _H2), bout.reshape(1, _OUT)
    cnts = [_counts_half(lax.slice_in_dim(x, i * _BSPLIT, (i + 1) * _BSPLIT))
            for i in range(_NSPLIT)]
    outs = [_mlp_call(c, embp, W1, b1r, W2, b2r, Wout, boutr) for c in cnts]
    return jnp.concatenate(outs, axis=0)


# speedup vs baseline: 1.0743x; 1.0743x over previous
import jax
import jax.numpy as jnp
from jax import lax
from jax.experimental import pallas as pl
from jax.experimental.pallas import tpu as pltpu
from jax.experimental.pallas import tpu_sc as plsc

_VOCAB = 1000
_VP = 1024
_EMB = 64
_B = 16384
_L = 200
_H1 = 256
_H2 = 256
_OUT = 128

_NW = 32
_ROWS_PER_W = _B // _NW
_CHUNK = 64
_NCHUNK = _ROWS_PER_W // _CHUNK
_JFULL = _L // 16


_WP = _VP // 4


def _make_counts_call(nrows):
    rows_per_w = nrows // _NW
    nchunk = rows_per_w // _CHUNK

    def body(x_hbm, cnt_hbm, x_v0, x_v1, cnt_v0, cnt_v1,
             sx0, sx1, sc0, sc1):
        wid = lax.axis_index("s") * 2 + lax.axis_index("c")
        lanes = lax.iota(jnp.int32, 16)
        tail_mask = lanes >= 8
        one16 = jnp.full((16,), 1, jnp.int32)
        zero16 = jnp.zeros((16,), jnp.int32)
        x_bufs, cnt_bufs = [x_v0, x_v1], [cnt_v0, cnt_v1]
        x_sems, cnt_sems = [sx0, sx1], [sc0, sc1]

        def scat(cv, xv, rvec, mask=None):
            col = jnp.bitwise_and(xv, 255)
            sh = jnp.right_shift(xv, 5) & 24
            val = jnp.left_shift(one16, sh)
            plsc.addupdate_scatter(cv, [rvec, col], val, mask=mask)

        def rowbase(c):
            return wid * rows_per_w + c * _CHUNK

        x_pend = [None, None]
        cnt_pend = [None, None]
        x_pend[0] = pltpu.async_copy(
            x_hbm.at[pl.ds(rowbase(0), _CHUNK)], x_bufs[0], x_sems[0])
        for c in range(nchunk):
            b = c % 2
            x_pend[b].wait()
            if c + 1 < nchunk:
                nb = (c + 1) % 2
                x_pend[nb] = pltpu.async_copy(
                    x_hbm.at[pl.ds(rowbase(c + 1), _CHUNK)],
                    x_bufs[nb], x_sems[nb])
            if cnt_pend[b] is not None:
                cnt_pend[b].wait()
            x_v, cnt_v = x_bufs[b], cnt_bufs[b]

            def zbody(r, carry):
                for j in range(_WP // 16):
                    cnt_v[r, pl.ds(j * 16, 16)] = zero16
                return carry

            lax.fori_loop(0, _CHUNK, zbody, 0)

            def rbody(rg, carry):
                rows = [rg * 4 + i for i in range(4)]
                rvecs = [jnp.full((16,), r, jnp.int32) for r in rows]
                for j in range(_JFULL):
                    xs = [x_v[r, pl.ds(j * 16, 16)] for r in rows]
                    for i in range(4):
                        scat(cnt_v, xs[i], rvecs[i])
                xts = [x_v[r, pl.ds(_L - 16, 16)] for r in rows]
                for i in range(4):
                    scat(cnt_v, xts[i], rvecs[i], mask=tail_mask)
                return carry

            lax.fori_loop(0, _CHUNK // 4, rbody, 0)

            cnt_pend[b] = pltpu.async_copy(
                cnt_v, cnt_hbm.at[pl.ds(rowbase(c), _CHUNK)], cnt_sems[b])
        for p in cnt_pend:
            if p is not None:
                p.wait()

    return pl.kernel(
        body,
        out_type=jax.ShapeDtypeStruct((nrows, _WP), jnp.int32),
        mesh=plsc.VectorSubcoreMesh(core_axis_name="c", subcore_axis_name="s"),
        scratch_types=[
            pltpu.VMEM((_CHUNK, _L), jnp.int32),
            pltpu.VMEM((_CHUNK, _L), jnp.int32),
            pltpu.VMEM((_CHUNK, _WP), jnp.int32),
            pltpu.VMEM((_CHUNK, _WP), jnp.int32),
            pltpu.SemaphoreType.DMA,
            pltpu.SemaphoreType.DMA,
            pltpu.SemaphoreType.DMA,
            pltpu.SemaphoreType.DMA,
        ],
        compiler_params=pltpu.CompilerParams(needs_layout_passes=False),
    )


_NSPLIT = 4
_BSPLIT = _B // _NSPLIT
_counts_half = _make_counts_call(_BSPLIT)

_BB = 512
_NBLK = _BSPLIT // _BB


def _mlp_body(cnt, embp, w1, b1, w2, b2, wout, bout, out, m1):
    @pl.when(pl.program_id(0) == 0)
    def _():
        m1[...] = (jnp.dot(embp[...], w1[...],
                           preferred_element_type=jnp.float32)
                   * (1.0 / _L)).astype(jnp.bfloat16)

    w = cnt[...]
    h = jnp.zeros((_BB, _H1), jnp.float32)
    for k in range(4):
        part = ((w >> (8 * k)) & 0xFF).astype(jnp.bfloat16)
        h = h + jnp.dot(part, m1[pl.ds(k * _WP, _WP), :],
                        preferred_element_type=jnp.float32)
    h = jnp.maximum(h + b1[...], 0.0)
    h = jnp.maximum(
        jnp.dot(h, w2[...], preferred_element_type=jnp.float32) + b2[...], 0.0)
    out[...] = jnp.dot(h, wout[...],
                       preferred_element_type=jnp.float32) + bout[...]


def _mlp_body_acc(cnt, embp, w1, b1, w2, b2, wout, bout, acc, out, m1):
    _mlp_body(cnt, embp, w1, b1, w2, b2, wout, bout, out, m1)


def _make_mlp_call(q):
    specs = [
        pl.BlockSpec((_BB, _WP), lambda i: (i, 0)),
        pl.BlockSpec((_VP, _EMB), lambda i: (0, 0)),
        pl.BlockSpec((_EMB, _H1), lambda i: (0, 0)),
        pl.BlockSpec((1, _H1), lambda i: (0, 0)),
        pl.BlockSpec((_H1, _H2), lambda i: (0, 0)),
        pl.BlockSpec((1, _H2), lambda i: (0, 0)),
        pl.BlockSpec((_H2, _OUT), lambda i: (0, 0)),
        pl.BlockSpec((1, _OUT), lambda i: (0, 0)),
    ]
    out_spec = pl.BlockSpec((_BB, _OUT), lambda i, q=q: (q * _NBLK + i, 0))
    out_shape = jax.ShapeDtypeStruct((_B, _OUT), jnp.float32)
    scratch = [pltpu.VMEM((_VP, _H1), jnp.bfloat16)]
    if q == 0:
        return pl.pallas_call(
            _mlp_body, grid=(_NBLK,), in_specs=specs, out_specs=out_spec,
            out_shape=out_shape, scratch_shapes=scratch)
    return pl.pallas_call(
        _mlp_body_acc, grid=(_NBLK,),
        in_specs=specs + [pl.BlockSpec(memory_space=pl.ANY)],
        out_specs=out_spec, out_shape=out_shape,
        input_output_aliases={8: 0}, scratch_shapes=scratch)


_mlp_calls = [_make_mlp_call(q) for q in range(_NSPLIT)]


def kernel(x, emb, W1, b1, W2, b2, Wout, bout):
    embp = jnp.pad(emb, ((0, _VP - _VOCAB), (0, 0)))
    b1r, b2r, boutr = b1.reshape(1, _H1), b2.reshape(1, _H2), bout.reshape(1, _OUT)
    cnts = [_counts_half(lax.slice_in_dim(x, i * _BSPLIT, (i + 1) * _BSPLIT))
            for i in range(_NSPLIT)]
    out = _mlp_calls[0](cnts[0], embp, W1, b1r, W2, b2r, Wout, boutr)
    for q in range(1, _NSPLIT):
        out = _mlp_calls[q](cnts[q], embp, W1, b1r, W2, b2r, Wout, boutr, out)
    return out


# speedup vs baseline: 1.1201x; 1.0426x over previous
import jax
import jax.numpy as jnp
from jax import lax
from jax.experimental import pallas as pl
from jax.experimental.pallas import tpu as pltpu
from jax.experimental.pallas import tpu_sc as plsc

_VOCAB = 1000
_VP = 1024
_EMB = 64
_B = 16384
_L = 200
_H1 = 256
_H2 = 256
_OUT = 128

_NW = 32
_ROWS_PER_W = _B // _NW
_CHUNK = 64
_NCHUNK = _ROWS_PER_W // _CHUNK
_JFULL = _L // 16


_WP = _VP // 4


def _make_counts_call(nrows):
    rows_per_w = nrows // _NW
    nchunk = rows_per_w // _CHUNK

    def body(x_hbm, cnt_hbm, x_v0, x_v1, cnt_v0, cnt_v1,
             sx0, sx1, sc0, sc1):
        wid = lax.axis_index("s") * 2 + lax.axis_index("c")
        lanes = lax.iota(jnp.int32, 16)
        tail_mask = lanes >= 8
        one16 = jnp.full((16,), 1, jnp.int32)
        zero16 = jnp.zeros((16,), jnp.int32)
        x_bufs, cnt_bufs = [x_v0, x_v1], [cnt_v0, cnt_v1]
        x_sems, cnt_sems = [sx0, sx1], [sc0, sc1]

        def scat(cv, xv, rvec, mask=None):
            col = jnp.bitwise_and(xv, 255)
            sh = jnp.right_shift(xv, 5) & 24
            val = jnp.left_shift(one16, sh)
            plsc.addupdate_scatter(cv, [rvec, col], val, mask=mask)

        def rowbase(c):
            return wid * rows_per_w + c * _CHUNK

        x_pend = [None, None]
        cnt_pend = [None, None]
        x_pend[0] = pltpu.async_copy(
            x_hbm.at[pl.ds(rowbase(0), _CHUNK)], x_bufs[0], x_sems[0])
        for c in range(nchunk):
            b = c % 2
            x_pend[b].wait()
            if c + 1 < nchunk:
                nb = (c + 1) % 2
                x_pend[nb] = pltpu.async_copy(
                    x_hbm.at[pl.ds(rowbase(c + 1), _CHUNK)],
                    x_bufs[nb], x_sems[nb])
            if cnt_pend[b] is not None:
                cnt_pend[b].wait()
            x_v, cnt_v = x_bufs[b], cnt_bufs[b]

            def zbody(r, carry):
                for j in range(_WP // 16):
                    cnt_v[r, pl.ds(j * 16, 16)] = zero16
                return carry

            lax.fori_loop(0, _CHUNK, zbody, 0)

            def rbody(rg, carry):
                rows = [rg * 4 + i for i in range(4)]
                rvecs = [jnp.full((16,), r, jnp.int32) for r in rows]
                for j in range(_JFULL):
                    xs = [x_v[r, pl.ds(j * 16, 16)] for r in rows]
                    for i in range(4):
                        scat(cnt_v, xs[i], rvecs[i])
                xts = [x_v[r, pl.ds(_L - 16, 16)] for r in rows]
                for i in range(4):
                    scat(cnt_v, xts[i], rvecs[i], mask=tail_mask)
                return carry

            lax.fori_loop(0, _CHUNK // 4, rbody, 0)

            cnt_pend[b] = pltpu.async_copy(
                cnt_v, cnt_hbm.at[pl.ds(rowbase(c), _CHUNK)], cnt_sems[b])
        for p in cnt_pend:
            if p is not None:
                p.wait()

    return pl.kernel(
        body,
        out_type=jax.ShapeDtypeStruct((nrows, _WP), jnp.int32),
        mesh=plsc.VectorSubcoreMesh(core_axis_name="c", subcore_axis_name="s"),
        scratch_types=[
            pltpu.VMEM((_CHUNK, _L), jnp.int32),
            pltpu.VMEM((_CHUNK, _L), jnp.int32),
            pltpu.VMEM((_CHUNK, _WP), jnp.int32),
            pltpu.VMEM((_CHUNK, _WP), jnp.int32),
            pltpu.SemaphoreType.DMA,
            pltpu.SemaphoreType.DMA,
            pltpu.SemaphoreType.DMA,
            pltpu.SemaphoreType.DMA,
        ],
        compiler_params=pltpu.CompilerParams(needs_layout_passes=False),
    )


_NSPLIT = 2
_BSPLIT = _B // _NSPLIT
_counts_half = _make_counts_call(_BSPLIT)

_BB = 1024
_NBLK = _BSPLIT // _BB


def _mlp_body(cnt, embp, w1, b1, w2, b2, wout, bout, out, m1):
    @pl.when(pl.program_id(0) == 0)
    def _():
        m1[...] = (jnp.dot(embp[...], w1[...],
                           preferred_element_type=jnp.float32)
                   * (1.0 / _L)).astype(jnp.bfloat16)

    w = cnt[...]
    h = jnp.zeros((_BB, _H1), jnp.float32)
    for k in range(4):
        part = ((w >> (8 * k)) & 0xFF).astype(jnp.bfloat16)
        h = h + jnp.dot(part, m1[pl.ds(k * _WP, _WP), :],
                        preferred_element_type=jnp.float32)
    h = jnp.maximum(h + b1[...], 0.0).astype(jnp.bfloat16)
    h = jnp.maximum(
        jnp.dot(h, w2[...].astype(jnp.bfloat16),
                preferred_element_type=jnp.float32) + b2[...], 0.0)
    out[...] = jnp.dot(h, wout[...],
                       preferred_element_type=jnp.float32) + bout[...]


def _mlp_body_acc(cnt, embp, w1, b1, w2, b2, wout, bout, acc, out, m1):
    _mlp_body(cnt, embp, w1, b1, w2, b2, wout, bout, out, m1)


def _make_mlp_call(q):
    specs = [
        pl.BlockSpec((_BB, _WP), lambda i: (i, 0)),
        pl.BlockSpec((_VP, _EMB), lambda i: (0, 0)),
        pl.BlockSpec((_EMB, _H1), lambda i: (0, 0)),
        pl.BlockSpec((1, _H1), lambda i: (0, 0)),
        pl.BlockSpec((_H1, _H2), lambda i: (0, 0)),
        pl.BlockSpec((1, _H2), lambda i: (0, 0)),
        pl.BlockSpec((_H2, _OUT), lambda i: (0, 0)),
        pl.BlockSpec((1, _OUT), lambda i: (0, 0)),
    ]
    out_spec = pl.BlockSpec((_BB, _OUT), lambda i, q=q: (q * _NBLK + i, 0))
    out_shape = jax.ShapeDtypeStruct((_B, _OUT), jnp.float32)
    scratch = [pltpu.VMEM((_VP, _H1), jnp.bfloat16)]
    if q == 0:
        return pl.pallas_call(
            _mlp_body, grid=(_NBLK,), in_specs=specs, out_specs=out_spec,
            out_shape=out_shape, scratch_shapes=scratch)
    return pl.pallas_call(
        _mlp_body_acc, grid=(_NBLK,),
        in_specs=specs + [pl.BlockSpec(memory_space=pl.ANY)],
        out_specs=out_spec, out_shape=out_shape,
        input_output_aliases={8: 0}, scratch_shapes=scratch)


_mlp_calls = [_make_mlp_call(q) for q in range(_NSPLIT)]


def kernel(x, emb, W1, b1, W2, b2, Wout, bout):
    embp = jnp.pad(emb, ((0, _VP - _VOCAB), (0, 0)))
    b1r, b2r, boutr = b1.reshape(1, _H1), b2.reshape(1, _H2), bout.reshape(1, _OUT)
    cnts = [_counts_half(lax.slice_in_dim(x, i * _BSPLIT, (i + 1) * _BSPLIT))
            for i in range(_NSPLIT)]
    out = _mlp_calls[0](cnts[0], embp, W1, b1r, W2, b2r, Wout, boutr)
    for q in range(1, _NSPLIT):
        out = _mlp_calls[q](cnts[q], embp, W1, b1r, W2, b2r, Wout, boutr, out)
    return out


# speedup vs baseline: 1.2132x; 1.0832x over previous
import jax
import jax.numpy as jnp
from jax import lax
from jax.experimental import pallas as pl
from jax.experimental.pallas import tpu as pltpu
from jax.experimental.pallas import tpu_sc as plsc

_VOCAB = 1000
_VP = 1024
_EMB = 64
_B = 16384
_L = 200
_H1 = 256
_H2 = 256
_OUT = 128

_NW = 32
_ROWS_PER_W = _B // _NW
_CHUNK = 64
_NCHUNK = _ROWS_PER_W // _CHUNK
_JFULL = _L // 16


_WP = _VP // 4


def _make_counts_call(row0, nrows):
    rows_per_w = nrows // _NW
    nchunk = rows_per_w // _CHUNK

    def body(x_hbm, cnt_hbm, x_v0, x_v1, cnt_v0, cnt_v1,
             sx0, sx1, sc0, sc1):
        wid = lax.axis_index("s") * 2 + lax.axis_index("c")
        lanes = lax.iota(jnp.int32, 16)
        tail_mask = lanes >= 8
        one16 = jnp.full((16,), 1, jnp.int32)
        zero16 = jnp.zeros((16,), jnp.int32)
        x_bufs, cnt_bufs = [x_v0, x_v1], [cnt_v0, cnt_v1]
        x_sems, cnt_sems = [sx0, sx1], [sc0, sc1]

        def scat(cv, xv, rvec, mask=None):
            col = jnp.bitwise_and(xv, 255)
            sh = jnp.right_shift(xv, 5) & 24
            val = jnp.left_shift(one16, sh)
            plsc.addupdate_scatter(cv, [rvec, col], val, mask=mask)

        def rowbase(c):
            return wid * rows_per_w + c * _CHUNK

        x_pend = [None, None]
        cnt_pend = [None, None]
        x_pend[0] = pltpu.async_copy(
            x_hbm.at[pl.ds(row0 + rowbase(0), _CHUNK)], x_bufs[0], x_sems[0])
        for c in range(nchunk):
            b = c % 2
            x_pend[b].wait()
            if c + 1 < nchunk:
                nb = (c + 1) % 2
                x_pend[nb] = pltpu.async_copy(
                    x_hbm.at[pl.ds(row0 + rowbase(c + 1), _CHUNK)],
                    x_bufs[nb], x_sems[nb])
            if cnt_pend[b] is not None:
                cnt_pend[b].wait()
            x_v, cnt_v = x_bufs[b], cnt_bufs[b]

            def zbody(r, carry):
                for j in range(_WP // 16):
                    cnt_v[r, pl.ds(j * 16, 16)] = zero16
                return carry

            lax.fori_loop(0, _CHUNK, zbody, 0)

            def rbody(rg, carry):
                rows = [rg * 4 + i for i in range(4)]
                rvecs = [jnp.full((16,), r, jnp.int32) for r in rows]
                for j in range(_JFULL):
                    xs = [x_v[r, pl.ds(j * 16, 16)] for r in rows]
                    for i in range(4):
                        scat(cnt_v, xs[i], rvecs[i])
                xts = [x_v[r, pl.ds(_L - 16, 16)] for r in rows]
                for i in range(4):
                    scat(cnt_v, xts[i], rvecs[i], mask=tail_mask)
                return carry

            lax.fori_loop(0, _CHUNK // 4, rbody, 0)

            cnt_pend[b] = pltpu.async_copy(
                cnt_v, cnt_hbm.at[pl.ds(rowbase(c), _CHUNK)], cnt_sems[b])
        for p in cnt_pend:
            if p is not None:
                p.wait()

    return pl.kernel(
        body,
        out_type=jax.ShapeDtypeStruct((nrows, _WP), jnp.int32),
        mesh=plsc.VectorSubcoreMesh(core_axis_name="c", subcore_axis_name="s"),
        scratch_types=[
            pltpu.VMEM((_CHUNK, _L), jnp.int32),
            pltpu.VMEM((_CHUNK, _L), jnp.int32),
            pltpu.VMEM((_CHUNK, _WP), jnp.int32),
            pltpu.VMEM((_CHUNK, _WP), jnp.int32),
            pltpu.SemaphoreType.DMA,
            pltpu.SemaphoreType.DMA,
            pltpu.SemaphoreType.DMA,
            pltpu.SemaphoreType.DMA,
        ],
        compiler_params=pltpu.CompilerParams(needs_layout_passes=False),
    )


_NSPLIT = 2
_BSPLIT = _B // _NSPLIT
_counts_calls = [_make_counts_call(i * _BSPLIT, _BSPLIT)
                 for i in range(_NSPLIT)]

_BB = 1024
_NBLK = _BSPLIT // _BB


def _mlp_body(cnt, embp, w1, b1, w2, b2, wout, bout, out, m1):
    @pl.when(pl.program_id(0) == 0)
    def _():
        m1[...] = (jnp.dot(embp[...], w1[...],
                           preferred_element_type=jnp.float32)
                   * (1.0 / _L)).astype(jnp.bfloat16)

    w = cnt[...]
    h = jnp.zeros((_BB, _H1), jnp.float32)
    for k in range(4):
        part = ((w >> (8 * k)) & 0xFF).astype(jnp.bfloat16)
        h = h + jnp.dot(part, m1[pl.ds(k * _WP, _WP), :],
                        preferred_element_type=jnp.float32)
    h = jnp.maximum(h + b1[...], 0.0).astype(jnp.bfloat16)
    h = jnp.maximum(
        jnp.dot(h, w2[...].astype(jnp.bfloat16),
                preferred_element_type=jnp.float32) + b2[...], 0.0)
    out[...] = jnp.dot(h, wout[...],
                       preferred_element_type=jnp.float32) + bout[...]


def _mlp_body_acc(cnt, embp, w1, b1, w2, b2, wout, bout, acc, out, m1):
    _mlp_body(cnt, embp, w1, b1, w2, b2, wout, bout, out, m1)


def _make_mlp_call(q):
    specs = [
        pl.BlockSpec((_BB, _WP), lambda i: (i, 0)),
        pl.BlockSpec((_VP, _EMB), lambda i: (0, 0)),
        pl.BlockSpec((_EMB, _H1), lambda i: (0, 0)),
        pl.BlockSpec((1, _H1), lambda i: (0, 0)),
        pl.BlockSpec((_H1, _H2), lambda i: (0, 0)),
        pl.BlockSpec((1, _H2), lambda i: (0, 0)),
        pl.BlockSpec((_H2, _OUT), lambda i: (0, 0)),
        pl.BlockSpec((1, _OUT), lambda i: (0, 0)),
    ]
    out_spec = pl.BlockSpec((_BB, _OUT), lambda i, q=q: (q * _NBLK + i, 0))
    out_shape = jax.ShapeDtypeStruct((_B, _OUT), jnp.float32)
    scratch = [pltpu.VMEM((_VP, _H1), jnp.bfloat16)]
    if q == 0:
        return pl.pallas_call(
            _mlp_body, grid=(_NBLK,), in_specs=specs, out_specs=out_spec,
            out_shape=out_shape, scratch_shapes=scratch)
    return pl.pallas_call(
        _mlp_body_acc, grid=(_NBLK,),
        in_specs=specs + [pl.BlockSpec(memory_space=pl.ANY)],
        out_specs=out_spec, out_shape=out_shape,
        input_output_aliases={8: 0}, scratch_shapes=scratch)


_mlp_calls = [_make_mlp_call(q) for q in range(_NSPLIT)]


def kernel(x, emb, W1, b1, W2, b2, Wout, bout):
    embp = jnp.pad(emb, ((0, _VP - _VOCAB), (0, 0)))
    b1r, b2r, boutr = b1.reshape(1, _H1), b2.reshape(1, _H2), bout.reshape(1, _OUT)
    cnts = [call(x) for call in _counts_calls]
    out = _mlp_calls[0](cnts[0], embp, W1, b1r, W2, b2r, Wout, boutr)
    for q in range(1, _NSPLIT):
        out = _mlp_calls[q](cnts[q], embp, W1, b1r, W2, b2r, Wout, boutr, out)
    return out


# speedup vs baseline: 1.2273x; 1.0116x over previous
import jax
import jax.numpy as jnp
from jax import lax
from jax.experimental import pallas as pl
from jax.experimental.pallas import tpu as pltpu
from jax.experimental.pallas import tpu_sc as plsc

_VOCAB = 1000
_VP = 1024
_EMB = 64
_B = 16384
_L = 200
_H1 = 256
_H2 = 256
_OUT = 128

_NW = 32
_ROWS_PER_W = _B // _NW
_CHUNK = 64
_NCHUNK = _ROWS_PER_W // _CHUNK
_JFULL = _L // 16


_WP = _VP // 4


def _make_counts_call(row0, nrows):
    rows_per_w = nrows // _NW
    nchunk = rows_per_w // _CHUNK

    def body(x_hbm, cnt_hbm, x_v0, x_v1, cnt_v0, cnt_v1,
             sx0, sx1, sc0, sc1):
        wid = lax.axis_index("s") * 2 + lax.axis_index("c")
        lanes = lax.iota(jnp.int32, 16)
        tail_mask = lanes >= 8
        one16 = jnp.full((16,), 1, jnp.int32)
        zero16 = jnp.zeros((16,), jnp.int32)
        x_bufs, cnt_bufs = [x_v0, x_v1], [cnt_v0, cnt_v1]
        x_sems, cnt_sems = [sx0, sx1], [sc0, sc1]

        def scat(cv, xv, rvec, mask=None):
            col = jnp.bitwise_and(xv, 255)
            sh = jnp.right_shift(xv, 5) & 24
            val = jnp.left_shift(one16, sh)
            plsc.addupdate_scatter(cv, [rvec, col], val, mask=mask)

        def rowbase(c):
            return wid * rows_per_w + c * _CHUNK

        x_pend = [None, None]
        cnt_pend = [None, None]
        x_pend[0] = pltpu.async_copy(
            x_hbm.at[pl.ds(row0 + rowbase(0), _CHUNK)], x_bufs[0], x_sems[0])
        for c in range(nchunk):
            b = c % 2
            x_pend[b].wait()
            if c + 1 < nchunk:
                nb = (c + 1) % 2
                x_pend[nb] = pltpu.async_copy(
                    x_hbm.at[pl.ds(row0 + rowbase(c + 1), _CHUNK)],
                    x_bufs[nb], x_sems[nb])
            if cnt_pend[b] is not None:
                cnt_pend[b].wait()
            x_v, cnt_v = x_bufs[b], cnt_bufs[b]

            def zbody(r, carry):
                for j in range(_WP // 16):
                    cnt_v[r, pl.ds(j * 16, 16)] = zero16
                return carry

            lax.fori_loop(0, _CHUNK, zbody, 0)

            def rbody(rg, carry):
                rows = [rg * 4 + i for i in range(4)]
                rvecs = [jnp.full((16,), r, jnp.int32) for r in rows]
                for j in range(_JFULL):
                    xs = [x_v[r, pl.ds(j * 16, 16)] for r in rows]
                    for i in range(4):
                        scat(cnt_v, xs[i], rvecs[i])
                xts = [x_v[r, pl.ds(_L - 16, 16)] for r in rows]
                for i in range(4):
                    scat(cnt_v, xts[i], rvecs[i], mask=tail_mask)
                return carry

            lax.fori_loop(0, _CHUNK // 4, rbody, 0)

            cnt_pend[b] = pltpu.async_copy(
                cnt_v, cnt_hbm.at[pl.ds(rowbase(c), _CHUNK)], cnt_sems[b])
        for p in cnt_pend:
            if p is not None:
                p.wait()

    return pl.kernel(
        body,
        out_type=jax.ShapeDtypeStruct((nrows, _WP), jnp.int32),
        mesh=plsc.VectorSubcoreMesh(core_axis_name="c", subcore_axis_name="s"),
        scratch_types=[
            pltpu.VMEM((_CHUNK, _L), jnp.int32),
            pltpu.VMEM((_CHUNK, _L), jnp.int32),
            pltpu.VMEM((_CHUNK, _WP), jnp.int32),
            pltpu.VMEM((_CHUNK, _WP), jnp.int32),
            pltpu.SemaphoreType.DMA,
            pltpu.SemaphoreType.DMA,
            pltpu.SemaphoreType.DMA,
            pltpu.SemaphoreType.DMA,
        ],
        compiler_params=pltpu.CompilerParams(needs_layout_passes=False),
    )


_NSPLIT = 2
_BSPLIT = _B // _NSPLIT
_counts_calls = [_make_counts_call(i * _BSPLIT, _BSPLIT)
                 for i in range(_NSPLIT)]

_BB = 2048
_NBLK = _BSPLIT // _BB


def _mlp_body(cnt, embp, w1, b1, w2, b2, wout, bout, out, m1):
    @pl.when(pl.program_id(0) == 0)
    def _():
        m1[...] = (jnp.dot(embp[...], w1[...],
                           preferred_element_type=jnp.float32)
                   * (1.0 / _L)).astype(jnp.bfloat16)

    w = cnt[...]
    h = jnp.zeros((_BB, _H1), jnp.float32)
    for k in range(4):
        part = ((w >> (8 * k)) & 0xFF).astype(jnp.bfloat16)
        h = h + jnp.dot(part, m1[pl.ds(k * _WP, _WP), :],
                        preferred_element_type=jnp.float32)
    h = jnp.maximum(h + b1[...], 0.0).astype(jnp.bfloat16)
    h = jnp.maximum(
        jnp.dot(h, w2[...].astype(jnp.bfloat16),
                preferred_element_type=jnp.float32) + b2[...], 0.0)
    out[...] = jnp.dot(h, wout[...],
                       preferred_element_type=jnp.float32) + bout[...]


def _mlp_body_acc(cnt, embp, w1, b1, w2, b2, wout, bout, acc, out, m1):
    _mlp_body(cnt, embp, w1, b1, w2, b2, wout, bout, out, m1)


def _make_mlp_call(q):
    specs = [
        pl.BlockSpec((_BB, _WP), lambda i: (i, 0)),
        pl.BlockSpec((_VP, _EMB), lambda i: (0, 0)),
        pl.BlockSpec((_EMB, _H1), lambda i: (0, 0)),
        pl.BlockSpec((1, _H1), lambda i: (0, 0)),
        pl.BlockSpec((_H1, _H2), lambda i: (0, 0)),
        pl.BlockSpec((1, _H2), lambda i: (0, 0)),
        pl.BlockSpec((_H2, _OUT), lambda i: (0, 0)),
        pl.BlockSpec((1, _OUT), lambda i: (0, 0)),
    ]
    out_spec = pl.BlockSpec((_BB, _OUT), lambda i, q=q: (q * _NBLK + i, 0))
    out_shape = jax.ShapeDtypeStruct((_B, _OUT), jnp.float32)
    scratch = [pltpu.VMEM((_VP, _H1), jnp.bfloat16)]
    if q == 0:
        return pl.pallas_call(
            _mlp_body, grid=(_NBLK,), in_specs=specs, out_specs=out_spec,
            out_shape=out_shape, scratch_shapes=scratch)
    return pl.pallas_call(
        _mlp_body_acc, grid=(_NBLK,),
        in_specs=specs + [pl.BlockSpec(memory_space=pl.ANY)],
        out_specs=out_spec, out_shape=out_shape,
        input_output_aliases={8: 0}, scratch_shapes=scratch)


_mlp_calls = [_make_mlp_call(q) for q in range(_NSPLIT)]


def kernel(x, emb, W1, b1, W2, b2, Wout, bout):
    embp = jnp.pad(emb, ((0, _VP - _VOCAB), (0, 0)))
    b1r, b2r, boutr = b1.reshape(1, _H1), b2.reshape(1, _H2), bout.reshape(1, _OUT)
    cnts = [call(x) for call in _counts_calls]
    out = _mlp_calls[0](cnts[0], embp, W1, b1r, W2, b2r, Wout, boutr)
    for q in range(1, _NSPLIT):
        out = _mlp_calls[q](cnts[q], embp, W1, b1r, W2, b2r, Wout, boutr, out)
    return out


# speedup vs baseline: 1.2680x; 1.0332x over previous
import jax
import jax.numpy as jnp
from jax import lax
from jax.experimental import pallas as pl
from jax.experimental.pallas import tpu as pltpu
from jax.experimental.pallas import tpu_sc as plsc

_VOCAB = 1000
_VP = 1024
_EMB = 64
_B = 16384
_L = 200
_H1 = 256
_H2 = 256
_OUT = 128

_NW = 32
_ROWS_PER_W = _B // _NW
_CHUNK = 64
_NCHUNK = _ROWS_PER_W // _CHUNK
_JFULL = _L // 16


_WP = _VP // 4


def _make_counts_call(row0, nrows):
    rows_per_w = nrows // _NW
    nchunk = rows_per_w // _CHUNK

    def body(x_hbm, cnt_hbm, x_v0, x_v1, cnt_v0, cnt_v1,
             sx0, sx1, sc0, sc1):
        wid = lax.axis_index("s") * 2 + lax.axis_index("c")
        lanes = lax.iota(jnp.int32, 16)
        tail_mask = lanes >= 8
        one16 = jnp.full((16,), 1, jnp.int32)
        zero16 = jnp.zeros((16,), jnp.int32)
        x_bufs, cnt_bufs = [x_v0, x_v1], [cnt_v0, cnt_v1]
        x_sems, cnt_sems = [sx0, sx1], [sc0, sc1]

        def scat(cv, xv, rvec, mask=None):
            col = jnp.bitwise_and(xv, 255)
            sh = jnp.right_shift(xv, 5) & 24
            val = jnp.left_shift(one16, sh)
            plsc.addupdate_scatter(cv, [rvec, col], val, mask=mask)

        def rowbase(c):
            return wid * rows_per_w + c * _CHUNK

        x_pend = [None, None]
        cnt_pend = [None, None]
        x_pend[0] = pltpu.async_copy(
            x_hbm.at[pl.ds(row0 + rowbase(0), _CHUNK)], x_bufs[0], x_sems[0])
        for c in range(nchunk):
            b = c % 2
            x_pend[b].wait()
            if c + 1 < nchunk:
                nb = (c + 1) % 2
                x_pend[nb] = pltpu.async_copy(
                    x_hbm.at[pl.ds(row0 + rowbase(c + 1), _CHUNK)],
                    x_bufs[nb], x_sems[nb])
            if cnt_pend[b] is not None:
                cnt_pend[b].wait()
            x_v, cnt_v = x_bufs[b], cnt_bufs[b]

            def zbody(r, carry):
                for j in range(_WP // 16):
                    cnt_v[r, pl.ds(j * 16, 16)] = zero16
                return carry

            lax.fori_loop(0, _CHUNK, zbody, 0)

            def rbody(rg, carry):
                rows = [rg * 8 + i for i in range(8)]
                rvecs = [jnp.full((16,), r, jnp.int32) for r in rows]
                for j in range(_JFULL):
                    xs = [x_v[r, pl.ds(j * 16, 16)] for r in rows]
                    for i in range(8):
                        scat(cnt_v, xs[i], rvecs[i])
                xts = [x_v[r, pl.ds(_L - 16, 16)] for r in rows]
                for i in range(8):
                    scat(cnt_v, xts[i], rvecs[i], mask=tail_mask)
                return carry

            lax.fori_loop(0, _CHUNK // 8, rbody, 0)

            cnt_pend[b] = pltpu.async_copy(
                cnt_v, cnt_hbm.at[pl.ds(rowbase(c), _CHUNK)], cnt_sems[b])
        for p in cnt_pend:
            if p is not None:
                p.wait()

    return pl.kernel(
        body,
        out_type=jax.ShapeDtypeStruct((nrows, _WP), jnp.int32),
        mesh=plsc.VectorSubcoreMesh(core_axis_name="c", subcore_axis_name="s"),
        scratch_types=[
            pltpu.VMEM((_CHUNK, _L), jnp.int32),
            pltpu.VMEM((_CHUNK, _L), jnp.int32),
            pltpu.VMEM((_CHUNK, _WP), jnp.int32),
            pltpu.VMEM((_CHUNK, _WP), jnp.int32),
            pltpu.SemaphoreType.DMA,
            pltpu.SemaphoreType.DMA,
            pltpu.SemaphoreType.DMA,
            pltpu.SemaphoreType.DMA,
        ],
        compiler_params=pltpu.CompilerParams(needs_layout_passes=False),
    )


_NSPLIT = 2
_BSPLIT = _B // _NSPLIT
_counts_calls = [_make_counts_call(i * _BSPLIT, _BSPLIT)
                 for i in range(_NSPLIT)]

_BB = 2048
_NBLK = _BSPLIT // _BB


def _mlp_body(cnt, embp, w1, b1, w2, b2, wout, bout, out, m1):
    @pl.when(pl.program_id(0) == 0)
    def _():
        m1[...] = (jnp.dot(embp[...], w1[...],
                           preferred_element_type=jnp.float32)
                   * (1.0 / _L)).astype(jnp.bfloat16)

    w = cnt[...]
    h = jnp.zeros((_BB, _H1), jnp.float32)
    for k in range(4):
        part = ((w >> (8 * k)) & 0xFF).astype(jnp.bfloat16)
        h = h + jnp.dot(part, m1[pl.ds(k * _WP, _WP), :],
                        preferred_element_type=jnp.float32)
    h = jnp.maximum(h + b1[...], 0.0).astype(jnp.bfloat16)
    h = jnp.maximum(
        jnp.dot(h, w2[...].astype(jnp.bfloat16),
                preferred_element_type=jnp.float32) + b2[...], 0.0)
    out[...] = jnp.dot(h, wout[...],
                       preferred_element_type=jnp.float32) + bout[...]


def _mlp_body_acc(cnt, embp, w1, b1, w2, b2, wout, bout, acc, out, m1):
    _mlp_body(cnt, embp, w1, b1, w2, b2, wout, bout, out, m1)


def _make_mlp_call(q):
    specs = [
        pl.BlockSpec((_BB, _WP), lambda i: (i, 0)),
        pl.BlockSpec((_VP, _EMB), lambda i: (0, 0)),
        pl.BlockSpec((_EMB, _H1), lambda i: (0, 0)),
        pl.BlockSpec((1, _H1), lambda i: (0, 0)),
        pl.BlockSpec((_H1, _H2), lambda i: (0, 0)),
        pl.BlockSpec((1, _H2), lambda i: (0, 0)),
        pl.BlockSpec((_H2, _OUT), lambda i: (0, 0)),
        pl.BlockSpec((1, _OUT), lambda i: (0, 0)),
    ]
    out_spec = pl.BlockSpec((_BB, _OUT), lambda i, q=q: (q * _NBLK + i, 0))
    out_shape = jax.ShapeDtypeStruct((_B, _OUT), jnp.float32)
    scratch = [pltpu.VMEM((_VP, _H1), jnp.bfloat16)]
    if q == 0:
        return pl.pallas_call(
            _mlp_body, grid=(_NBLK,), in_specs=specs, out_specs=out_spec,
            out_shape=out_shape, scratch_shapes=scratch)
    return pl.pallas_call(
        _mlp_body_acc, grid=(_NBLK,),
        in_specs=specs + [pl.BlockSpec(memory_space=pl.ANY)],
        out_specs=out_spec, out_shape=out_shape,
        input_output_aliases={8: 0}, scratch_shapes=scratch)


_mlp_calls = [_make_mlp_call(q) for q in range(_NSPLIT)]


def kernel(x, emb, W1, b1, W2, b2, Wout, bout):
    embp = jnp.pad(emb, ((0, _VP - _VOCAB), (0, 0)))
    b1r, b2r, boutr = b1.reshape(1, _H1), b2.reshape(1, _H2), bout.reshape(1, _OUT)
    cnts = [call(x) for call in _counts_calls]
    out = _mlp_calls[0](cnts[0], embp, W1, b1r, W2, b2r, Wout, boutr)
    for q in range(1, _NSPLIT):
        out = _mlp_calls[q](cnts[q], embp, W1, b1r, W2, b2r, Wout, boutr, out)
    return out
